# Initial kernel scaffold; baseline (speedup 1.0000x reference)
import jax
import jax.numpy as jnp
from jax.experimental import pallas as pl


def kernel(x, edge_index, emb_table, W1, b1, W2, b2):
    raise NotImplementedError("write your pallas kernel here")


# speedup vs baseline: 12.1559x; 12.1559x over previous
import functools

import jax
import jax.numpy as jnp
from jax import lax
from jax.experimental import pallas as pl
from jax.experimental.pallas import tpu as pltpu
from jax.experimental.pallas import tpu_sc as plsc

N = 100000
E = 1600000
L = 16
GPR = 128
KG = 8
NGRP = 12800
EP = NGRP * GPR
NBLK = NGRP // KG
NW = 32
ITERS = NBLK // NW
ICH = 200
NCH = N // ICH
ITER_CH = (NCH + 15) // 16
ACC_ROWS = N + 16

_mesh = lambda: plsc.VectorSubcoreMesh(core_axis_name="c", subcore_axis_name="s")


def _fill(buf, val):
    def body(r, carry):
        buf[r] = jnp.full((L,), val, jnp.float32)
        return carry
    lax.fori_loop(0, buf.shape[0], body, 0)


def _make_deg_kernel():
    @functools.partial(
        pl.kernel,
        out_type=jax.ShapeDtypeStruct((2, N, L), jnp.float32),
        mesh=_mesh(),
        compiler_params=pltpu.CompilerParams(use_tc_tiling_on_sc=False),
        scratch_types=[
            pltpu.VMEM_SHARED((ACC_ROWS, L), jnp.float32),
            pltpu.VMEM((KG, GPR), jnp.int32),
            pltpu.VMEM((GPR, L), jnp.float32),
            pltpu.VMEM((ICH, L), jnp.float32),
        ],
    )
    def deg_kernel(dst_hbm, out, acc, didx, ones_v, zbuf):
        cid = lax.axis_index("c")
        sid = lax.axis_index("s")
        wid = sid * 2 + cid
        _fill(ones_v, 1.0)
        _fill(zbuf, 0.0)
        for k in range(ITER_CH):
            t = sid + k * 16
            @pl.when(t < NCH)
            def _(t=t):
                pltpu.sync_copy(zbuf, acc.at[pl.ds(t * ICH, ICH)])
        plsc.subcore_barrier()

        def eb(it, carry):
            blk = wid + it * NW
            pltpu.sync_copy(dst_hbm.at[pl.ds(blk * KG, KG)], didx)
            for j in range(KG):
                pltpu.sync_copy(ones_v, acc.at[didx.at[j]], add=True)
            return carry
        lax.fori_loop(0, ITERS, eb, 0)
        plsc.subcore_barrier()
        for k in range(ITER_CH):
            t = sid + k * 16
            @pl.when(t < NCH)
            def _(t=t):
                r = t * ICH
                pltpu.sync_copy(acc.at[pl.ds(r, ICH)], zbuf)
                pltpu.sync_copy(zbuf, out.at[cid, pl.ds(r, ICH)])
    return deg_kernel


def _make_agg_kernel(nchunk):
    scratch = [
        pltpu.VMEM_SHARED((ACC_ROWS, L), jnp.float32),
        pltpu.VMEM((KG, GPR), jnp.int32),
        pltpu.VMEM((KG, GPR), jnp.int32),
        pltpu.VMEM((KG, GPR, L), jnp.float32),
        pltpu.VMEM((ICH, L), jnp.float32),
        pltpu.VMEM((ICH, L), jnp.float32),
        pltpu.SemaphoreType.DMA,
    ]

    @functools.partial(
        pl.kernel,
        out_type=jax.ShapeDtypeStruct((2 * nchunk, N, L), jnp.float32),
        mesh=_mesh(),
        compiler_params=pltpu.CompilerParams(use_tc_tiling_on_sc=False),
        scratch_types=scratch,
    )
    def agg_kernel(src_hbm, dst_hbm, *rest):
        tables = rest[:nchunk]
        out = rest[nchunk]
        acc, sidx, didx, rows, ibuf, zbuf, sem = rest[nchunk + 1:]
        cid = lax.axis_index("c")
        sid = lax.axis_index("s")
        wid = sid * 2 + cid
        _fill(zbuf, 0.0)

        for c in range(nchunk):
            table = tables[c]
            for k in range(ITER_CH):
                t = sid + k * 16
                @pl.when((t < NCH) & (cid == 0))
                def _(t=t, table=table):
                    r = t * ICH
                    pltpu.sync_copy(table.at[pl.ds(r, ICH)], ibuf)
                    pltpu.sync_copy(ibuf, acc.at[pl.ds(r, ICH)])
                @pl.when((t < NCH) & (cid != 0))
                def _(t=t):
                    pltpu.sync_copy(zbuf, acc.at[pl.ds(t * ICH, ICH)])
            plsc.subcore_barrier()

            def eb(it, carry, table=table):
                blk = wid + it * NW
                g0 = blk * KG
                pltpu.sync_copy(src_hbm.at[pl.ds(g0, KG)], sidx)
                pltpu.sync_copy(dst_hbm.at[pl.ds(g0, KG)], didx)
                descs = [
                    pltpu.async_copy(table.at[sidx.at[j]], rows.at[j], sem)
                    for j in range(KG)
                ]
                for d_ in descs:
                    d_.wait()
                for j in range(KG):
                    pltpu.sync_copy(rows.at[j], acc.at[didx.at[j]], add=True)
                return carry
            lax.fori_loop(0, ITERS, eb, 0)
            plsc.subcore_barrier()

            for k in range(ITER_CH):
                t = sid + k * 16
                @pl.when(t < NCH)
                def _(t=t, c=c):
                    r = t * ICH
                    pltpu.sync_copy(acc.at[pl.ds(r, ICH)], ibuf)
                    pltpu.sync_copy(ibuf, out.at[cid * nchunk + c, pl.ds(r, ICH)])
            plsc.subcore_barrier()
    return agg_kernel


_deg_kernel = _make_deg_kernel()
_agg4 = _make_agg_kernel(4)
_agg2 = _make_agg_kernel(2)

RB = 1000


def _tc_b_body(h_ref, w1_ref, dp_ref, dis_ref, g0_ref, g1_ref, g2_ref, g3_ref):
    deg = 1.0 + dp_ref[0, :, 0:1] + dp_ref[1, :, 0:1]
    dis = lax.rsqrt(deg)
    g = jnp.dot(h_ref[...], w1_ref[...], preferred_element_type=jnp.float32) * dis
    dis_ref[...] = dis
    for c, ref in enumerate((g0_ref, g1_ref, g2_ref, g3_ref)):
        ref[...] = g[:, c * L:(c + 1) * L]


def _tc_b(h, W1, degp):
    grid = N // RB
    return pl.pallas_call(
        _tc_b_body,
        grid=(grid,),
        in_specs=[
            pl.BlockSpec((RB, 32), lambda i: (i, 0)),
            pl.BlockSpec((32, 64), lambda i: (0, 0)),
            pl.BlockSpec((2, RB, L), lambda i: (0, i, 0)),
        ],
        out_specs=[
            pl.BlockSpec((RB, 1), lambda i: (i, 0)),
            pl.BlockSpec((RB, L), lambda i: (i, 0)),
            pl.BlockSpec((RB, L), lambda i: (i, 0)),
            pl.BlockSpec((RB, L), lambda i: (i, 0)),
            pl.BlockSpec((RB, L), lambda i: (i, 0)),
        ],
        out_shape=[
            jax.ShapeDtypeStruct((N, 1), jnp.float32),
            jax.ShapeDtypeStruct((N, L), jnp.float32),
            jax.ShapeDtypeStruct((N, L), jnp.float32),
            jax.ShapeDtypeStruct((N, L), jnp.float32),
            jax.ShapeDtypeStruct((N, L), jnp.float32),
        ],
    )(h, W1, degp)


def _tc_d_body(dis_ref, p_ref, b1_ref, w2_ref, q0_ref, q1_ref):
    dis = dis_ref[...]
    hcs = []
    for c in range(4):
        pre = dis * (p_ref[c] + p_ref[4 + c]) + b1_ref[0, c * L:(c + 1) * L]
        hcs.append(jnp.maximum(pre, 0.0))
    for d, ref in enumerate((q0_ref, q1_ref)):
        acc = jnp.zeros((RB, L), jnp.float32)
        for c in range(4):
            acc += jnp.dot(hcs[c], w2_ref[c * L:(c + 1) * L, d * L:(d + 1) * L],
                           preferred_element_type=jnp.float32)
        ref[...] = acc * dis


def _tc_d(dis, p, b1, W2):
    grid = N // RB
    return pl.pallas_call(
        _tc_d_body,
        grid=(grid,),
        in_specs=[
            pl.BlockSpec((RB, 1), lambda i: (i, 0)),
            pl.BlockSpec((8, RB, L), lambda i: (0, i, 0)),
            pl.BlockSpec((1, 64), lambda i: (0, 0)),
            pl.BlockSpec((64, 32), lambda i: (0, 0)),
        ],
        out_specs=[
            pl.BlockSpec((RB, L), lambda i: (i, 0)),
            pl.BlockSpec((RB, L), lambda i: (i, 0)),
        ],
        out_shape=[
            jax.ShapeDtypeStruct((N, L), jnp.float32),
            jax.ShapeDtypeStruct((N, L), jnp.float32),
        ],
    )(dis, p, b1, W2)


def _tc_f_body(dis_ref, q_ref, b2_ref, o_ref):
    dis = dis_ref[...]
    parts = [dis * (q_ref[d] + q_ref[2 + d]) + b2_ref[0, d * L:(d + 1) * L]
             for d in range(2)]
    o_ref[...] = jnp.concatenate(parts, axis=1)


def _tc_f(dis, q, b2):
    grid = N // RB
    return pl.pallas_call(
        _tc_f_body,
        grid=(grid,),
        in_specs=[
            pl.BlockSpec((RB, 1), lambda i: (i, 0)),
            pl.BlockSpec((4, RB, L), lambda i: (0, i, 0)),
            pl.BlockSpec((1, 32), lambda i: (0, 0)),
        ],
        out_specs=pl.BlockSpec((RB, 32), lambda i: (i, 0)),
        out_shape=jax.ShapeDtypeStruct((N, 32), jnp.float32),
    )(dis, q, b2)


def kernel(x, edge_index, emb_table, W1, b1, W2, b2):
    h = jnp.take(emb_table, x[:, 0], axis=0)
    npad = EP - E
    src2 = jnp.concatenate(
        [edge_index[0], jnp.zeros((npad,), jnp.int32)]).reshape(NGRP, GPR)
    dst2 = jnp.concatenate(
        [edge_index[1], jnp.full((npad,), N, jnp.int32)]).reshape(NGRP, GPR)
    degp = _deg_kernel(dst2)
    dis, g0, g1, g2, g3 = _tc_b(h, W1, degp)
    p = _agg4(src2, dst2, g0, g1, g2, g3)
    q0, q1 = _tc_d(dis, p, b1.reshape(1, 64), W2)
    q = _agg2(src2, dst2, q0, q1)
    return _tc_f(dis, q, b2.reshape(1, 32))


# speedup vs baseline: 13.7521x; 1.1313x over previous
import functools

import jax
import jax.numpy as jnp
from jax import lax
from jax.experimental import pallas as pl
from jax.experimental.pallas import tpu as pltpu
from jax.experimental.pallas import tpu_sc as plsc

N = 100000
E = 1600000
L = 16
GPR = 128
KG = 4
NGRP = 12800
EP = NGRP * GPR
NBLK = NGRP // KG
NW = 32
ITERS = NBLK // NW
ICH = 200
NCH = N // ICH
ITER_CH = (NCH + 15) // 16
ACC_ROWS = N + 16

_mesh = lambda: plsc.VectorSubcoreMesh(core_axis_name="c", subcore_axis_name="s")


def _fill(buf, val):
    def body(r, carry):
        buf[r] = jnp.full((L,), val, jnp.float32)
        return carry
    lax.fori_loop(0, buf.shape[0], body, 0)


def _make_deg_kernel():
    @functools.partial(
        pl.kernel,
        out_type=jax.ShapeDtypeStruct((2, N, L), jnp.float32),
        mesh=_mesh(),
        compiler_params=pltpu.CompilerParams(use_tc_tiling_on_sc=False),
        scratch_types=[
            pltpu.VMEM_SHARED((ACC_ROWS, L), jnp.float32),
            pltpu.VMEM((KG, 2, GPR), jnp.int32),
            pltpu.VMEM((GPR, L), jnp.float32),
            pltpu.VMEM((ICH, L), jnp.float32),
        ],
    )
    def deg_kernel(e_hbm, out, acc, didx, ones_v, zbuf):
        cid = lax.axis_index("c")
        sid = lax.axis_index("s")
        wid = sid * 2 + cid
        _fill(ones_v, 1.0)
        _fill(zbuf, 0.0)
        for k in range(ITER_CH):
            t = sid + k * 16
            @pl.when(t < NCH)
            def _(t=t):
                pltpu.sync_copy(zbuf, acc.at[pl.ds(t * ICH, ICH)])
        plsc.subcore_barrier()

        def eb(it, carry):
            blk = wid + it * NW
            pltpu.sync_copy(e_hbm.at[pl.ds(blk * KG, KG)], didx)
            for j in range(KG):
                pltpu.sync_copy(ones_v, acc.at[didx.at[j, 1]], add=True)
            return carry
        lax.fori_loop(0, ITERS, eb, 0)
        plsc.subcore_barrier()
        for k in range(ITER_CH):
            t = sid + k * 16
            @pl.when(t < NCH)
            def _(t=t):
                r = t * ICH
                pltpu.sync_copy(acc.at[pl.ds(r, ICH)], zbuf)
                pltpu.sync_copy(zbuf, out.at[cid, pl.ds(r, ICH)])
    return deg_kernel


def _make_agg_kernel(nchunk):
    scratch = [
        pltpu.VMEM_SHARED((ACC_ROWS, L), jnp.float32),
        pltpu.VMEM((KG, 2, GPR), jnp.int32),
        pltpu.VMEM((KG, 2, GPR), jnp.int32),
        pltpu.VMEM((KG, GPR, L), jnp.float32),
        pltpu.VMEM((KG, GPR, L), jnp.float32),
        pltpu.VMEM((ICH, L), jnp.float32),
        pltpu.VMEM((ICH, L), jnp.float32),
        pltpu.SemaphoreType.DMA,
        pltpu.SemaphoreType.DMA,
    ]

    @functools.partial(
        pl.kernel,
        out_type=jax.ShapeDtypeStruct((2 * nchunk, N, L), jnp.float32),
        mesh=_mesh(),
        compiler_params=pltpu.CompilerParams(use_tc_tiling_on_sc=False),
        scratch_types=scratch,
    )
    def agg_kernel(e_hbm, *rest):
        tables = rest[:nchunk]
        out = rest[nchunk]
        acc, eidx0, eidx1, rows0, rows1, ibuf, zbuf, sem0, sem1 = rest[nchunk + 1:]
        eidx = (eidx0, eidx1)
        rows = (rows0, rows1)
        sems = (sem0, sem1)
        cid = lax.axis_index("c")
        sid = lax.axis_index("s")
        wid = sid * 2 + cid
        _fill(zbuf, 0.0)

        for c in range(nchunk):
            table = tables[c]
            for k in range(ITER_CH):
                t = sid + k * 16
                @pl.when((t < NCH) & (cid == 0))
                def _(t=t, table=table):
                    r = t * ICH
                    pltpu.sync_copy(table.at[pl.ds(r, ICH)], ibuf)
                    pltpu.sync_copy(ibuf, acc.at[pl.ds(r, ICH)])
                @pl.when((t < NCH) & (cid != 0))
                def _(t=t):
                    pltpu.sync_copy(zbuf, acc.at[pl.ds(t * ICH, ICH)])
            plsc.subcore_barrier()

            def fire(s, i, table=table):
                blk = wid + i * NW
                pltpu.sync_copy(e_hbm.at[pl.ds(blk * KG, KG)], eidx[s])
                for j in range(KG):
                    pltpu.async_copy(table.at[eidx[s].at[j, 0]],
                                     rows[s].at[j], sems[s])

            def wait_gathers(s, table=table):
                for j in range(KG):
                    pltpu.make_async_copy(table.at[eidx[s].at[j, 0]],
                                          rows[s].at[j], sems[s]).wait()

            fire(0, 0)

            def eb2(it2, carry):
                for b in (0, 1):
                    i = it2 * 2 + b
                    p, q = b, 1 - b
                    @pl.when(i + 1 < ITERS)
                    def _(q=q, i=i):
                        fire(q, i + 1)
                    wait_gathers(p)
                    for j in range(KG):
                        pltpu.sync_copy(rows[p].at[j],
                                        acc.at[eidx[p].at[j, 1]], add=True)
                return carry
            lax.fori_loop(0, ITERS // 2, eb2, 0)
            plsc.subcore_barrier()

            for k in range(ITER_CH):
                t = sid + k * 16
                @pl.when(t < NCH)
                def _(t=t, c=c):
                    r = t * ICH
                    pltpu.sync_copy(acc.at[pl.ds(r, ICH)], ibuf)
                    pltpu.sync_copy(ibuf, out.at[cid * nchunk + c, pl.ds(r, ICH)])
            plsc.subcore_barrier()
    return agg_kernel


_deg_kernel = _make_deg_kernel()
_agg4 = _make_agg_kernel(4)
_agg2 = _make_agg_kernel(2)

RB = 1000


def _tc_b_body(h_ref, w1_ref, dp_ref, dis_ref, g0_ref, g1_ref, g2_ref, g3_ref):
    deg = 1.0 + dp_ref[0, :, 0:1] + dp_ref[1, :, 0:1]
    dis = lax.rsqrt(deg)
    g = jnp.dot(h_ref[...], w1_ref[...], preferred_element_type=jnp.float32) * dis
    dis_ref[...] = dis
    for c, ref in enumerate((g0_ref, g1_ref, g2_ref, g3_ref)):
        ref[...] = g[:, c * L:(c + 1) * L]


def _tc_b(h, W1, degp):
    grid = N // RB
    return pl.pallas_call(
        _tc_b_body,
        grid=(grid,),
        in_specs=[
            pl.BlockSpec((RB, 32), lambda i: (i, 0)),
            pl.BlockSpec((32, 64), lambda i: (0, 0)),
            pl.BlockSpec((2, RB, L), lambda i: (0, i, 0)),
        ],
        out_specs=[
            pl.BlockSpec((RB, 1), lambda i: (i, 0)),
            pl.BlockSpec((RB, L), lambda i: (i, 0)),
            pl.BlockSpec((RB, L), lambda i: (i, 0)),
            pl.BlockSpec((RB, L), lambda i: (i, 0)),
            pl.BlockSpec((RB, L), lambda i: (i, 0)),
        ],
        out_shape=[
            jax.ShapeDtypeStruct((N, 1), jnp.float32),
            jax.ShapeDtypeStruct((N, L), jnp.float32),
            jax.ShapeDtypeStruct((N, L), jnp.float32),
            jax.ShapeDtypeStruct((N, L), jnp.float32),
            jax.ShapeDtypeStruct((N, L), jnp.float32),
        ],
    )(h, W1, degp)


def _tc_d_body(dis_ref, p_ref, b1_ref, w2_ref, q0_ref, q1_ref):
    dis = dis_ref[...]
    hcs = []
    for c in range(4):
        pre = dis * (p_ref[c] + p_ref[4 + c]) + b1_ref[0, c * L:(c + 1) * L]
        hcs.append(jnp.maximum(pre, 0.0))
    for d, ref in enumerate((q0_ref, q1_ref)):
        acc = jnp.zeros((RB, L), jnp.float32)
        for c in range(4):
            acc += jnp.dot(hcs[c], w2_ref[c * L:(c + 1) * L, d * L:(d + 1) * L],
                           preferred_element_type=jnp.float32)
        ref[...] = acc * dis


def _tc_d(dis, p, b1, W2):
    grid = N // RB
    return pl.pallas_call(
        _tc_d_body,
        grid=(grid,),
        in_specs=[
            pl.BlockSpec((RB, 1), lambda i: (i, 0)),
            pl.BlockSpec((8, RB, L), lambda i: (0, i, 0)),
            pl.BlockSpec((1, 64), lambda i: (0, 0)),
            pl.BlockSpec((64, 32), lambda i: (0, 0)),
        ],
        out_specs=[
            pl.BlockSpec((RB, L), lambda i: (i, 0)),
            pl.BlockSpec((RB, L), lambda i: (i, 0)),
        ],
        out_shape=[
            jax.ShapeDtypeStruct((N, L), jnp.float32),
            jax.ShapeDtypeStruct((N, L), jnp.float32),
        ],
    )(dis, p, b1, W2)


def _tc_f_body(dis_ref, q_ref, b2_ref, o_ref):
    dis = dis_ref[...]
    parts = [dis * (q_ref[d] + q_ref[2 + d]) + b2_ref[0, d * L:(d + 1) * L]
             for d in range(2)]
    o_ref[...] = jnp.concatenate(parts, axis=1)


def _tc_f(dis, q, b2):
    grid = N // RB
    return pl.pallas_call(
        _tc_f_body,
        grid=(grid,),
        in_specs=[
            pl.BlockSpec((RB, 1), lambda i: (i, 0)),
            pl.BlockSpec((4, RB, L), lambda i: (0, i, 0)),
            pl.BlockSpec((1, 32), lambda i: (0, 0)),
        ],
        out_specs=pl.BlockSpec((RB, 32), lambda i: (i, 0)),
        out_shape=jax.ShapeDtypeStruct((N, 32), jnp.float32),
    )(dis, q, b2)


def kernel(x, edge_index, emb_table, W1, b1, W2, b2):
    h = jnp.take(emb_table, x[:, 0], axis=0)
    npad = EP - E
    src2 = jnp.concatenate(
        [edge_index[0], jnp.zeros((npad,), jnp.int32)]).reshape(NGRP, GPR)
    dst2 = jnp.concatenate(
        [edge_index[1], jnp.full((npad,), N, jnp.int32)]).reshape(NGRP, GPR)
    e2 = jnp.stack([src2, dst2], axis=1)
    degp = _deg_kernel(e2)
    dis, g0, g1, g2, g3 = _tc_b(h, W1, degp)
    p = _agg4(e2, g0, g1, g2, g3)
    q0, q1 = _tc_d(dis, p, b1.reshape(1, 64), W2)
    q = _agg2(e2, q0, q1)
    return _tc_f(dis, q, b2.reshape(1, 32))
